# baseline (speedup 1.0000x reference)
import functools

import jax
import jax.numpy as jnp
from jax.experimental import pallas as pl

_POS_BLOCK = 256


def _body(x_ref, w_ref, o_ref):
    w = w_ref[...]
    ss = jnp.sum(w * w, axis=1, keepdims=True)
    norm = jnp.sqrt(ss)
    scale = jnp.where(norm > 1.0, 1.0 / (norm + 1e-7), 1.0)
    o_ref[...] = x_ref[...] + (w * scale)[None]


@functools.partial(jax.jit, static_argnames=())
def kernel(x, pos_emb_weight):
    batch, num_pos, feat = x.shape
    np_blocks = num_pos // _POS_BLOCK
    b_blk = 4
    return pl.pallas_call(
        _body,
        grid=(np_blocks, batch // b_blk),
        in_specs=[
            pl.BlockSpec((b_blk, _POS_BLOCK, feat), lambda p, b: (b, p, 0)),
            pl.BlockSpec((_POS_BLOCK, feat), lambda p, b: (p, 0)),
        ],
        out_specs=pl.BlockSpec((b_blk, _POS_BLOCK, feat), lambda p, b: (b, p, 0)),
        out_shape=jax.ShapeDtypeStruct(x.shape, x.dtype),
    )(x, pos_emb_weight)


# speedup vs baseline: 1.0375x; 1.0375x over previous
import functools

import jax
import jax.numpy as jnp
from jax.experimental import pallas as pl

_POS_BLOCK = 512


def _body(x_ref, w_ref, o_ref):
    w = w_ref[...]
    ss = jnp.sum(w * w, axis=1, keepdims=True)
    norm = jnp.sqrt(ss)
    scale = jnp.where(norm > 1.0, 1.0 / (norm + 1e-7), 1.0)
    o_ref[...] = x_ref[...] + (w * scale)[None]


@jax.jit
def kernel(x, pos_emb_weight):
    batch, num_pos, feat = x.shape
    np_blocks = num_pos // _POS_BLOCK
    return pl.pallas_call(
        _body,
        grid=(np_blocks,),
        in_specs=[
            pl.BlockSpec((batch, _POS_BLOCK, feat), lambda p: (0, p, 0)),
            pl.BlockSpec((_POS_BLOCK, feat), lambda p: (p, 0)),
        ],
        out_specs=pl.BlockSpec((batch, _POS_BLOCK, feat), lambda p: (0, p, 0)),
        out_shape=jax.ShapeDtypeStruct(x.shape, x.dtype),
    )(x, pos_emb_weight)
